# baseline (speedup 1.0000x reference)
import jax
import jax.numpy as jnp
from jax import lax
from jax.experimental import pallas as pl
from jax.experimental.pallas import tpu as pltpu
from jax.experimental.pallas import tpu_sc as plsc

VOCAB = 100000
FUNC = 50000
EMB = 64
B = 16384
K = 28
NC = 2
NS = 16
NW = NC * NS
ROWS_PER = B // NW
C = 4
NCH = ROWS_PER // C
NBUF = 2
CK = C * K
CE = C * 6
KP = 32


def _sc_pred_kernel(ridx_hbm, eidx_hbm, fidx_hbm, emb_w_hbm, emb_f_hbm,
                    emb_r_hbm, out_hbm, ridx_v, eidx_v, fidx_v, rbufs, ebufs,
                    fbufs, pred_v, sems):
    wid = lax.axis_index("s") * NC + lax.axis_index("c")

    pltpu.sync_copy(ridx_hbm.at[wid], ridx_v)
    pltpu.sync_copy(eidx_hbm.at[wid], eidx_v)
    pltpu.sync_copy(fidx_hbm.at[wid], fidx_v)

    def fire(j, b):
        pltpu.async_copy(emb_r_hbm.at[ridx_v.at[j]], rbufs[b], sems[b])
        pltpu.async_copy(emb_w_hbm.at[eidx_v.at[j]], ebufs[b], sems[b])
        pltpu.async_copy(emb_f_hbm.at[fidx_v.at[j]], fbufs[b], sems[b])

    def drain(j, b):
        pltpu.make_async_copy(emb_r_hbm.at[ridx_v.at[j]], rbufs[b],
                              sems[b]).wait()
        pltpu.make_async_copy(emb_w_hbm.at[eidx_v.at[j]], ebufs[b],
                              sems[b]).wait()
        pltpu.make_async_copy(emb_f_hbm.at[fidx_v.at[j]], fbufs[b],
                              sems[b]).wait()

    lane = lax.iota(jnp.int32, 16)

    def compute(j, rbuf, ebuf, fbuf):
        third = jnp.float32(1.0 / 3.0)
        half = jnp.float32(0.5)
        for i in range(C):
            vs = []
            for d in range(8):
                f = fbuf[i, pl.ds(16 * d, 16)]
                if d < 4:
                    prev = ebuf[6 * i + 0, pl.ds(16 * d, 16)]
                    nxt = ebuf[6 * i + 3, pl.ds(16 * d, 16)]
                    v = (f + prev + nxt) * third
                else:
                    dd = d - 4
                    s = (ebuf[6 * i + 1, pl.ds(16 * dd, 16)]
                         + ebuf[6 * i + 2, pl.ds(16 * dd, 16)]
                         + ebuf[6 * i + 4, pl.ds(16 * dd, 16)]
                         + ebuf[6 * i + 5, pl.ds(16 * dd, 16)])
                    v = (f + s * half) * third
                vs.append(v)
            for g in range(2):
                nk = 16 if g == 0 else K - 16
                svs = []
                for m in range(nk):
                    r = K * i + 16 * g + m
                    p = [rbuf[r, pl.ds(16 * d, 16)] * vs[d] for d in range(8)]
                    s4 = [p[0] + p[1], p[2] + p[3], p[4] + p[5], p[6] + p[7]]
                    svs.append((s4[0] + s4[1]) + (s4[2] + s4[3]))
                svs = [jnp.full((16,), jnp.sum(a), jnp.float32) for a in svs]
                svs = svs + [svs[-1]] * (16 - len(svs))
                while len(svs) > 1:
                    s = 16 // len(svs)
                    mask = (lane % (2 * s)) < s
                    svs = [jnp.where(mask, svs[2 * t], svs[2 * t + 1])
                           for t in range(len(svs) // 2)]
                pred_v[j, pl.ds(KP * i + 16 * g, 16)] = svs[0]

    for b in range(NBUF):
        fire(b, b)

    def body(jj, _):
        j = NBUF * jj
        for b in range(NBUF):
            drain(j + b, b)
            compute(j + b, rbufs[b], ebufs[b], fbufs[b])

            @pl.when(j + b + NBUF < NCH)
            def _f():
                fire(j + b + NBUF, b)

        return _

    lax.fori_loop(0, NCH // NBUF, body, None)
    pltpu.sync_copy(pred_v, out_hbm.at[wid])


def _sc_pred(ridx, eidx, fidx, emb_w, emb_f_w, emb_r_w):
    mesh = plsc.VectorSubcoreMesh(core_axis_name="c", subcore_axis_name="s",
                                  num_cores=NC, num_subcores=NS)
    return pl.kernel(
        _sc_pred_kernel,
        out_type=jax.ShapeDtypeStruct((NW, NCH, C * KP), jnp.float32),
        mesh=mesh,
        compiler_params=pltpu.CompilerParams(needs_layout_passes=False,
                                             use_tc_tiling_on_sc=False),
        scratch_types=[
            pltpu.VMEM((NCH, CK), jnp.int32),
            pltpu.VMEM((NCH, CE), jnp.int32),
            pltpu.VMEM((NCH, C), jnp.int32),
            [pltpu.VMEM((CK, 2 * EMB), jnp.float32) for _ in range(NBUF)],
            [pltpu.VMEM((CE, EMB), jnp.float32) for _ in range(NBUF)],
            [pltpu.VMEM((C, 2 * EMB), jnp.float32) for _ in range(NBUF)],
            pltpu.VMEM((NCH, C * KP), jnp.float32),
            [pltpu.SemaphoreType.DMA for _ in range(NBUF)],
        ],
    )(ridx, eidx, fidx, emb_w, emb_f_w, emb_r_w)


def _loss_body(pred_ref, out_ref):
    x = pred_ref[...]
    cols = x.shape[1]
    flat = (lax.broadcasted_iota(jnp.int32, x.shape, 0) * cols
            + lax.broadcasted_iota(jnp.int32, x.shape, 1))
    k = flat % KP
    p = jax.nn.sigmoid(x)
    eps = 1e-7
    p = jnp.clip(p, eps, 1.0 - eps)
    term = jnp.where(k < 3, jnp.log(p),
                     jnp.where(k < K, jnp.log(1.0 - p), 0.0))
    out_ref[0, 0] = -jnp.sum(term) / (B * K)


def _tc_loss(pred2d):
    return pl.pallas_call(
        _loss_body,
        out_shape=jax.ShapeDtypeStruct((1, 1), jnp.float32),
        out_specs=pl.BlockSpec(memory_space=pltpu.SMEM),
    )(pred2d)


@jax.jit
def kernel(inp, pos, neg, emb_w, emb_f_w, emb_r_w):
    ridx = jnp.concatenate([pos, neg], axis=1).reshape(NW, NCH, CK)
    eidx = inp[:, 1:7].reshape(NW, NCH, CE)
    fidx = inp[:, 0].reshape(NW, NCH, C)
    pred = _sc_pred(ridx.astype(jnp.int32), eidx.astype(jnp.int32),
                    fidx.astype(jnp.int32), emb_w, emb_f_w, emb_r_w)
    pred2d = pred.reshape(B * KP // 128, 128)
    loss = _tc_loss(pred2d)
    return loss[0, 0]


# speedup vs baseline: 1.4781x; 1.4781x over previous
import jax
import jax.numpy as jnp
from jax import lax
from jax.experimental import pallas as pl
from jax.experimental.pallas import tpu as pltpu
from jax.experimental.pallas import tpu_sc as plsc

VOCAB = 100000
FUNC = 50000
EMB = 64
B = 16384
K = 28
NC = 2
NS = 16
NW = NC * NS
ROWS_PER = B // NW
C = 4
NCH = ROWS_PER // C
NBUF = 2
CK = C * K
CE = C * 6
KP = 32


def _sc_pred_kernel(ridx_hbm, eidx_hbm, fidx_hbm, emb_w_hbm, emb_f_hbm,
                    emb_r_hbm, out_hbm, ridx_v, eidx_v, fidx_v, rbufs, ebufs,
                    fbufs, pred_v, sems):
    wid = lax.axis_index("s") * NC + lax.axis_index("c")

    pltpu.sync_copy(ridx_hbm.at[wid], ridx_v)
    pltpu.sync_copy(eidx_hbm.at[wid], eidx_v)
    pltpu.sync_copy(fidx_hbm.at[wid], fidx_v)

    def fire(j, b):
        pltpu.async_copy(emb_r_hbm.at[ridx_v.at[j]], rbufs[b], sems[b])
        pltpu.async_copy(emb_w_hbm.at[eidx_v.at[j]], ebufs[b], sems[b])
        pltpu.async_copy(emb_f_hbm.at[fidx_v.at[j]], fbufs[b], sems[b])

    def drain(j, b):
        pltpu.make_async_copy(emb_r_hbm.at[ridx_v.at[j]], rbufs[b],
                              sems[b]).wait()
        pltpu.make_async_copy(emb_w_hbm.at[eidx_v.at[j]], ebufs[b],
                              sems[b]).wait()
        pltpu.make_async_copy(emb_f_hbm.at[fidx_v.at[j]], fbufs[b],
                              sems[b]).wait()

    lane = lax.iota(jnp.int32, 16)

    def compute(j, rbuf, ebuf, fbuf):
        third = jnp.float32(1.0 / 3.0)
        half = jnp.float32(0.5)
        for i in range(C):
            vs = []
            for d in range(8):
                f = fbuf[i, pl.ds(16 * d, 16)]
                if d < 4:
                    prev = ebuf[6 * i + 0, pl.ds(16 * d, 16)]
                    nxt = ebuf[6 * i + 3, pl.ds(16 * d, 16)]
                    v = (f + prev + nxt) * third
                else:
                    dd = d - 4
                    s = (ebuf[6 * i + 1, pl.ds(16 * dd, 16)]
                         + ebuf[6 * i + 2, pl.ds(16 * dd, 16)]
                         + ebuf[6 * i + 4, pl.ds(16 * dd, 16)]
                         + ebuf[6 * i + 5, pl.ds(16 * dd, 16)])
                    v = (f + s * half) * third
                vs.append(v)
            def kbody(k, carry):
                vec0, vec1 = carry
                acc = rbuf[K * i + k, pl.ds(0, 16)] * vs[0]
                for d in range(1, 8):
                    acc = acc + rbuf[K * i + k, pl.ds(16 * d, 16)] * vs[d]
                sv = jnp.full((16,), jnp.sum(acc), jnp.float32)
                lo = k < 16
                hit = lane == jnp.where(lo, k, k - 16)
                vec0 = jnp.where(lo & hit, sv, vec0)
                vec1 = jnp.where(jnp.logical_not(lo) & hit, sv, vec1)
                return vec0, vec1

            z = jnp.zeros((16,), jnp.float32)
            vec0, vec1 = lax.fori_loop(0, K, kbody, (z, z))
            pred_v[j, pl.ds(KP * i, 16)] = vec0
            pred_v[j, pl.ds(KP * i + 16, 16)] = vec1

    for b in range(NBUF):
        fire(b, b)

    def body(jj, _):
        j = NBUF * jj
        for b in range(NBUF):
            drain(j + b, b)
            compute(j + b, rbufs[b], ebufs[b], fbufs[b])

            @pl.when(j + b + NBUF < NCH)
            def _f():
                fire(j + b + NBUF, b)

        return _

    lax.fori_loop(0, NCH // NBUF, body, None)
    pltpu.sync_copy(pred_v, out_hbm.at[wid])


def _sc_pred(ridx, eidx, fidx, emb_w, emb_f_w, emb_r_w):
    mesh = plsc.VectorSubcoreMesh(core_axis_name="c", subcore_axis_name="s",
                                  num_cores=NC, num_subcores=NS)
    return pl.kernel(
        _sc_pred_kernel,
        out_type=jax.ShapeDtypeStruct((NW, NCH, C * KP), jnp.float32),
        mesh=mesh,
        compiler_params=pltpu.CompilerParams(needs_layout_passes=False,
                                             use_tc_tiling_on_sc=False),
        scratch_types=[
            pltpu.VMEM((NCH, CK), jnp.int32),
            pltpu.VMEM((NCH, CE), jnp.int32),
            pltpu.VMEM((NCH, C), jnp.int32),
            [pltpu.VMEM((CK, 2 * EMB), jnp.float32) for _ in range(NBUF)],
            [pltpu.VMEM((CE, EMB), jnp.float32) for _ in range(NBUF)],
            [pltpu.VMEM((C, 2 * EMB), jnp.float32) for _ in range(NBUF)],
            pltpu.VMEM((NCH, C * KP), jnp.float32),
            [pltpu.SemaphoreType.DMA for _ in range(NBUF)],
        ],
    )(ridx, eidx, fidx, emb_w, emb_f_w, emb_r_w)


def _loss_body(pred_ref, out_ref):
    x = pred_ref[...]
    cols = x.shape[1]
    flat = (lax.broadcasted_iota(jnp.int32, x.shape, 0) * cols
            + lax.broadcasted_iota(jnp.int32, x.shape, 1))
    k = flat % KP
    p = jax.nn.sigmoid(x)
    eps = 1e-7
    p = jnp.clip(p, eps, 1.0 - eps)
    term = jnp.where(k < 3, jnp.log(p),
                     jnp.where(k < K, jnp.log(1.0 - p), 0.0))
    out_ref[0, 0] = -jnp.sum(term) / (B * K)


def _tc_loss(pred2d):
    return pl.pallas_call(
        _loss_body,
        out_shape=jax.ShapeDtypeStruct((1, 1), jnp.float32),
        out_specs=pl.BlockSpec(memory_space=pltpu.SMEM),
    )(pred2d)


@jax.jit
def kernel(inp, pos, neg, emb_w, emb_f_w, emb_r_w):
    ridx = jnp.concatenate([pos, neg], axis=1).reshape(NW, NCH, CK)
    eidx = inp[:, 1:7].reshape(NW, NCH, CE)
    fidx = inp[:, 0].reshape(NW, NCH, C)
    pred = _sc_pred(ridx.astype(jnp.int32), eidx.astype(jnp.int32),
                    fidx.astype(jnp.int32), emb_w, emb_f_w, emb_r_w)
    pred2d = pred.reshape(B * KP // 128, 128)
    loss = _tc_loss(pred2d)
    return loss[0, 0]


# speedup vs baseline: 1.5686x; 1.0612x over previous
import jax
import jax.numpy as jnp
from jax import lax
from jax.experimental import pallas as pl
from jax.experimental.pallas import tpu as pltpu
from jax.experimental.pallas import tpu_sc as plsc

VOCAB = 100000
FUNC = 50000
EMB = 64
B = 16384
K = 28
NC = 2
NS = 16
NW = NC * NS
ROWS_PER = B // NW
C = 4
NCH = ROWS_PER // C
NBUF = 4
CK = C * K
CE = C * 6
KP = 32


def _sc_pred_kernel(ridx_hbm, eidx_hbm, fidx_hbm, emb_w_hbm, emb_f_hbm,
                    emb_r_hbm, out_hbm, ridx_v, eidx_v, fidx_v, rbufs, ebufs,
                    fbufs, pred_v, sems):
    wid = lax.axis_index("s") * NC + lax.axis_index("c")

    pltpu.sync_copy(ridx_hbm.at[wid], ridx_v)
    pltpu.sync_copy(eidx_hbm.at[wid], eidx_v)
    pltpu.sync_copy(fidx_hbm.at[wid], fidx_v)

    def fire(j, b):
        pltpu.async_copy(emb_r_hbm.at[ridx_v.at[j]], rbufs[b], sems[b])
        pltpu.async_copy(emb_w_hbm.at[eidx_v.at[j]], ebufs[b], sems[b])
        pltpu.async_copy(emb_f_hbm.at[fidx_v.at[j]], fbufs[b], sems[b])

    def drain(j, b):
        pltpu.make_async_copy(emb_r_hbm.at[ridx_v.at[j]], rbufs[b],
                              sems[b]).wait()
        pltpu.make_async_copy(emb_w_hbm.at[eidx_v.at[j]], ebufs[b],
                              sems[b]).wait()
        pltpu.make_async_copy(emb_f_hbm.at[fidx_v.at[j]], fbufs[b],
                              sems[b]).wait()

    lane = lax.iota(jnp.int32, 16)

    def compute(j, rbuf, ebuf, fbuf):
        third = jnp.float32(1.0 / 3.0)
        half = jnp.float32(0.5)
        for i in range(C):
            vs = []
            for d in range(8):
                f = fbuf[i, pl.ds(16 * d, 16)]
                if d < 4:
                    prev = ebuf[6 * i + 0, pl.ds(16 * d, 16)]
                    nxt = ebuf[6 * i + 3, pl.ds(16 * d, 16)]
                    v = (f + prev + nxt) * third
                else:
                    dd = d - 4
                    s = (ebuf[6 * i + 1, pl.ds(16 * dd, 16)]
                         + ebuf[6 * i + 2, pl.ds(16 * dd, 16)]
                         + ebuf[6 * i + 4, pl.ds(16 * dd, 16)]
                         + ebuf[6 * i + 5, pl.ds(16 * dd, 16)])
                    v = (f + s * half) * third
                vs.append(v)
            def kbody(k, carry):
                vec0, vec1 = carry
                acc = rbuf[K * i + k, pl.ds(0, 16)] * vs[0]
                for d in range(1, 8):
                    acc = acc + rbuf[K * i + k, pl.ds(16 * d, 16)] * vs[d]
                sv = jnp.full((16,), jnp.sum(acc), jnp.float32)
                lo = k < 16
                hit = lane == jnp.where(lo, k, k - 16)
                vec0 = jnp.where(lo & hit, sv, vec0)
                vec1 = jnp.where(jnp.logical_not(lo) & hit, sv, vec1)
                return vec0, vec1

            z = jnp.zeros((16,), jnp.float32)
            vec0, vec1 = lax.fori_loop(0, K, kbody, (z, z))
            pred_v[j, pl.ds(KP * i, 16)] = vec0
            pred_v[j, pl.ds(KP * i + 16, 16)] = vec1

    for b in range(NBUF):
        fire(b, b)

    def body(jj, _):
        j = NBUF * jj
        for b in range(NBUF):
            drain(j + b, b)
            compute(j + b, rbufs[b], ebufs[b], fbufs[b])

            @pl.when(j + b + NBUF < NCH)
            def _f():
                fire(j + b + NBUF, b)

        return _

    lax.fori_loop(0, NCH // NBUF, body, None)
    pltpu.sync_copy(pred_v, out_hbm.at[wid])


def _sc_pred(ridx, eidx, fidx, emb_w, emb_f_w, emb_r_w):
    mesh = plsc.VectorSubcoreMesh(core_axis_name="c", subcore_axis_name="s",
                                  num_cores=NC, num_subcores=NS)
    return pl.kernel(
        _sc_pred_kernel,
        out_type=jax.ShapeDtypeStruct((NW, NCH, C * KP), jnp.float32),
        mesh=mesh,
        compiler_params=pltpu.CompilerParams(needs_layout_passes=False,
                                             use_tc_tiling_on_sc=False),
        scratch_types=[
            pltpu.VMEM((NCH, CK), jnp.int32),
            pltpu.VMEM((NCH, CE), jnp.int32),
            pltpu.VMEM((NCH, C), jnp.int32),
            [pltpu.VMEM((CK, 2 * EMB), jnp.float32) for _ in range(NBUF)],
            [pltpu.VMEM((CE, EMB), jnp.float32) for _ in range(NBUF)],
            [pltpu.VMEM((C, 2 * EMB), jnp.float32) for _ in range(NBUF)],
            pltpu.VMEM((NCH, C * KP), jnp.float32),
            [pltpu.SemaphoreType.DMA for _ in range(NBUF)],
        ],
    )(ridx, eidx, fidx, emb_w, emb_f_w, emb_r_w)


def _loss_body(pred_ref, out_ref):
    x = pred_ref[...]
    cols = x.shape[1]
    flat = (lax.broadcasted_iota(jnp.int32, x.shape, 0) * cols
            + lax.broadcasted_iota(jnp.int32, x.shape, 1))
    k = flat % KP
    p = jax.nn.sigmoid(x)
    eps = 1e-7
    p = jnp.clip(p, eps, 1.0 - eps)
    term = jnp.where(k < 3, jnp.log(p),
                     jnp.where(k < K, jnp.log(1.0 - p), 0.0))
    out_ref[0, 0] = -jnp.sum(term) / (B * K)


def _tc_loss(pred2d):
    return pl.pallas_call(
        _loss_body,
        out_shape=jax.ShapeDtypeStruct((1, 1), jnp.float32),
        out_specs=pl.BlockSpec(memory_space=pltpu.SMEM),
    )(pred2d)


@jax.jit
def kernel(inp, pos, neg, emb_w, emb_f_w, emb_r_w):
    ridx = jnp.concatenate([pos, neg], axis=1).reshape(NW, NCH, CK)
    eidx = inp[:, 1:7].reshape(NW, NCH, CE)
    fidx = inp[:, 0].reshape(NW, NCH, C)
    pred = _sc_pred(ridx.astype(jnp.int32), eidx.astype(jnp.int32),
                    fidx.astype(jnp.int32), emb_w, emb_f_w, emb_r_w)
    pred2d = pred.reshape(B * KP // 128, 128)
    loss = _tc_loss(pred2d)
    return loss[0, 0]
